# Initial kernel scaffold; baseline (speedup 1.0000x reference)
import jax
import jax.numpy as jnp
from jax.experimental import pallas as pl


def kernel(x, edge_index, edge_attr, W1, b1, W2, b2):
    raise NotImplementedError("write your pallas kernel here")


# speedup vs baseline: 9.7599x; 9.7599x over previous
import functools

import jax
import jax.numpy as jnp
from jax import lax
from jax.experimental import pallas as pl
from jax.experimental.pallas import tpu as pltpu
from jax.experimental.pallas import tpu_sc as plsc

N = 10000
E = 320000
D = 128

NC = 2
NS = 16
L = 16

NP = 10240
RPS = NP // NS
CH = 128
EPAD = 323584
EPT = EPAD // (NC * NS)
NCH = EPT // CH

_vec_mesh = plsc.VectorSubcoreMesh(core_axis_name="c", subcore_axis_name="s")


def _deg_body(dst_hbm, w_hbm, degp_hbm, dst_v, w_v, deg_v):
    c = lax.axis_index("c")
    s = lax.axis_index("s")
    tid = c * NS + s
    base = tid * EPT

    zero = jnp.zeros((L,), jnp.float32)

    @pl.loop(0, NP // L, unroll=8)
    def _(i):
        deg_v[pl.ds(i * L, L)] = zero

    pltpu.sync_copy(dst_hbm.at[pl.ds(base, EPT)], dst_v)
    pltpu.sync_copy(w_hbm.at[pl.ds(base, EPT)], w_v)

    @pl.loop(0, EPT // L, unroll=4)
    def _(i):
        idx = dst_v[pl.ds(i * L, L)]
        wv = w_v[pl.ds(i * L, L)]
        plsc.addupdate_scatter(deg_v, [idx], wv)

    pltpu.sync_copy(deg_v, degp_hbm.at[tid])


@functools.partial(
    pl.kernel,
    out_type=jax.ShapeDtypeStruct((NC * NS, NP), jnp.float32),
    mesh=_vec_mesh,
    compiler_params=pltpu.CompilerParams(needs_layout_passes=False),
    scratch_types=[
        pltpu.VMEM((EPT,), jnp.int32),
        pltpu.VMEM((EPT,), jnp.float32),
        pltpu.VMEM((NP,), jnp.float32),
    ],
)
def _deg_kernel(dst_hbm, w_hbm, degp_hbm, dst_v, w_v, deg_v):
    _deg_body(dst_hbm, w_hbm, degp_hbm, dst_v, w_v, deg_v)


def _prop_body(g_hbm, src_hbm, dst_hbm, w_hbm, acc0_hbm, acc1_hbm,
               src_v, dst_v, w_v, rows_v, acc_sh, gsem, ssem):
    c = lax.axis_index("c")
    s = lax.axis_index("s")
    tid = c * NS + s
    base = tid * EPT

    zero = jnp.zeros((L,), jnp.float32)

    @pl.loop(0, CH)
    def _(i):
        for j in range(D // L):
            rows_v[i, pl.ds(j * L, L)] = zero

    for k in range(RPS // CH):
        pltpu.sync_copy(rows_v, acc_sh.at[pl.ds(s * RPS + k * CH, CH)])
    plsc.subcore_barrier()

    @pl.loop(0, NCH)
    def _(i):
        eb = base + i * CH
        pltpu.sync_copy(src_hbm.at[pl.ds(eb, CH)], src_v)
        pltpu.sync_copy(dst_hbm.at[pl.ds(eb, CH)], dst_v)
        pltpu.sync_copy(w_hbm.at[pl.ds(eb, CH)], w_v)
        pltpu.async_copy(g_hbm.at[src_v], rows_v, gsem).wait()

        @pl.loop(0, CH // L)
        def _(k):
            w16 = w_v[pl.ds(k * L, L)]
            for t in range(L):
                e = k * L + t
                ws = w16[t]
                for j in range(D // L):
                    sl = pl.ds(j * L, L)
                    rows_v[e, sl] = rows_v[e, sl] * ws

        pltpu.async_copy(rows_v, acc_sh.at[dst_v], ssem, add=True).wait()

    plsc.subcore_barrier()
    rsl = pl.ds(s * RPS, RPS)

    @pl.when(c == 0)
    def _():
        pltpu.sync_copy(acc_sh.at[rsl], acc0_hbm.at[rsl])

    @pl.when(c == 1)
    def _():
        pltpu.sync_copy(acc_sh.at[rsl], acc1_hbm.at[rsl])


@functools.partial(
    pl.kernel,
    out_type=(
        jax.ShapeDtypeStruct((NP, D), jnp.float32),
        jax.ShapeDtypeStruct((NP, D), jnp.float32),
    ),
    mesh=_vec_mesh,
    scratch_types=[
        pltpu.VMEM((CH,), jnp.int32),
        pltpu.VMEM((CH,), jnp.int32),
        pltpu.VMEM((CH,), jnp.float32),
        pltpu.VMEM((CH, D), jnp.float32),
        pltpu.VMEM_SHARED((NP, D), jnp.float32),
        pltpu.SemaphoreType.DMA,
        pltpu.SemaphoreType.DMA,
    ],
)
def _prop_kernel(g_hbm, src_hbm, dst_hbm, w_hbm, acc0_hbm, acc1_hbm,
                 src_v, dst_v, w_v, rows_v, acc_sh, gsem, ssem):
    _prop_body(g_hbm, src_hbm, dst_hbm, w_hbm, acc0_hbm, acc1_hbm,
               src_v, dst_v, w_v, rows_v, acc_sh, gsem, ssem)


RB = 1024


def _dinv_block(degp):
    deg = jnp.sum(degp, axis=0) + 1.0
    return jnp.where(deg > 0, lax.rsqrt(jnp.maximum(deg, 1e-12)), 0.0)


def _mm(a, b):
    return lax.dot_general(a, b, (((1,), (0,)), ((), ())),
                           preferred_element_type=jnp.float32,
                           precision=lax.Precision.HIGHEST)


def _tc_g1_body(degp_ref, x_ref, w1_ref, g1_ref):
    dinv = _dinv_block(degp_ref[...])
    g1_ref[...] = _mm(x_ref[...], w1_ref[...]) * dinv[:, None]


def _tc_g2_body(degp_ref, a0_ref, a1_ref, g1_ref, b1_ref, w2_ref, g2_ref):
    dinv = _dinv_block(degp_ref[...])
    acc = a0_ref[...] + a1_ref[...] + g1_ref[...]
    x2 = jnp.maximum(acc * dinv[:, None] + b1_ref[...], 0.0)
    g2_ref[...] = _mm(x2, w2_ref[...]) * dinv[:, None]


def _tc_out_body(degp_ref, a0_ref, a1_ref, g2_ref, b2_ref, out_ref):
    dinv = _dinv_block(degp_ref[...])
    acc = a0_ref[...] + a1_ref[...] + g2_ref[...]
    out_ref[...] = acc * dinv[:, None] + b2_ref[...]


_degp_spec = pl.BlockSpec((NC * NS, RB), lambda i: (0, i))
_rows_spec = pl.BlockSpec((RB, D), lambda i: (i, 0))
_mat_spec = pl.BlockSpec((D, D), lambda i: (0, 0))
_bias_spec = pl.BlockSpec((1, D), lambda i: (0, 0))
_grid = (NP // RB,)

_tc_g1 = pl.pallas_call(
    _tc_g1_body,
    grid=_grid,
    in_specs=[_degp_spec, _rows_spec, _mat_spec],
    out_specs=_rows_spec,
    out_shape=jax.ShapeDtypeStruct((NP, D), jnp.float32),
)

_tc_g2 = pl.pallas_call(
    _tc_g2_body,
    grid=_grid,
    in_specs=[_degp_spec, _rows_spec, _rows_spec, _rows_spec, _bias_spec,
              _mat_spec],
    out_specs=_rows_spec,
    out_shape=jax.ShapeDtypeStruct((NP, D), jnp.float32),
)

_tc_out = pl.pallas_call(
    _tc_out_body,
    grid=_grid,
    in_specs=[_degp_spec, _rows_spec, _rows_spec, _rows_spec, _bias_spec],
    out_specs=_rows_spec,
    out_shape=jax.ShapeDtypeStruct((NP, D), jnp.float32),
)


def kernel(x, edge_index, edge_attr, W1, b1, W2, b2):
    src = edge_index[0]
    dst = edge_index[1]
    pad = EPAD - E
    pad_idx = jnp.full((pad,), NP - 1, jnp.int32)
    src_p = jnp.concatenate([src, pad_idx])
    dst_p = jnp.concatenate([dst, pad_idx])
    w_p = jnp.concatenate([edge_attr, jnp.zeros((pad,), jnp.float32)])
    x_p = jnp.pad(x, ((0, NP - N), (0, 0)))

    degp = _deg_kernel(dst_p, w_p)
    g1 = _tc_g1(degp, x_p, W1)
    a0, a1 = _prop_kernel(g1, src_p, dst_p, w_p)
    g2 = _tc_g2(degp, a0, a1, g1, b1.reshape(1, D), W2)
    a0b, a1b = _prop_kernel(g2, src_p, dst_p, w_p)
    out = _tc_out(degp, a0b, a1b, g2, b2.reshape(1, D))
    return out[:N]
